# Initial kernel scaffold; baseline (speedup 1.0000x reference)
import jax
import jax.numpy as jnp
from jax.experimental import pallas as pl


def kernel(nodes, edge_index, edge_attr, params):
    raise NotImplementedError("write your pallas kernel here")


# speedup vs baseline: 1.1177x; 1.1177x over previous
import functools

import jax
import jax.numpy as jnp
from jax import lax
from jax.experimental import pallas as pl
from jax.experimental.pallas import tpu as pltpu
from jax.experimental.pallas import tpu_sc as plsc

N_NODES = 10000
N_EDGES = 320000
HID = 128
DW = 256
SW = 128
NC = 2
NS = 16
NW = NC * NS
EPW = N_EDGES // NW
B = 40
NCHUNK = EPW // B
ZR = 40
NZC = N_NODES // ZR

_f32 = jnp.float32


def _dot(a, b):
    return jnp.dot(a, b, preferred_element_type=_f32)


def _proj_body(x_ref, ws_ref, wd_ref, ps_ref, pd_ref):
    x = x_ref[...]
    ps_ref[...] = _dot(x, ws_ref[...])
    pd_ref[...] = _dot(x, wd_ref[...])


def _proj(x, wsrc, wdst):
    bn = 1000
    grid = (N_NODES // bn,)
    return pl.pallas_call(
        _proj_body,
        grid=grid,
        in_specs=[
            pl.BlockSpec((bn, HID), lambda i: (i, 0)),
            pl.BlockSpec((HID, DW), lambda i: (0, 0)),
            pl.BlockSpec((HID, DW), lambda i: (0, 0)),
        ],
        out_specs=[
            pl.BlockSpec((bn, DW), lambda i: (i, 0)),
            pl.BlockSpec((bn, DW), lambda i: (i, 0)),
        ],
        out_shape=[
            jax.ShapeDtypeStruct((N_NODES, DW), _f32),
            jax.ShapeDtypeStruct((N_NODES, DW), _f32),
        ],
    )(x, wsrc, wdst)


def _edge_pre_body(ea_ref, c_ref, b_ref, out_ref):
    out_ref[...] = _dot(ea_ref[...], c_ref[...]) + b_ref[...]


def _edge_pre(ea, ccat, b1cat):
    be = 2000
    grid = (N_EDGES // be,)
    return pl.pallas_call(
        _edge_pre_body,
        grid=grid,
        in_specs=[
            pl.BlockSpec((be, 4), lambda i: (i, 0)),
            pl.BlockSpec((4, DW), lambda i: (0, 0)),
            pl.BlockSpec((1, DW), lambda i: (0, 0)),
        ],
        out_specs=pl.BlockSpec((be, DW), lambda i: (i, 0)),
        out_shape=jax.ShapeDtypeStruct((N_EDGES, DW), _f32),
    )(ea, ccat, b1cat)


def _edge_pre2_body(h_ref, w2_ref, b2_ref, c_ref, b_ref, out_ref):
    ea = _dot(h_ref[...], w2_ref[...]) + b2_ref[...]
    out_ref[...] = _dot(ea, c_ref[...]) + b_ref[...]


def _edge_pre2(hedge, w2e, b2e, ccat, b1cat):
    be = 2000
    grid = (N_EDGES // be,)
    return pl.pallas_call(
        _edge_pre2_body,
        grid=grid,
        in_specs=[
            pl.BlockSpec((be, HID), lambda i: (i, 0)),
            pl.BlockSpec((HID, 4), lambda i: (0, 0)),
            pl.BlockSpec((1, 4), lambda i: (0, 0)),
            pl.BlockSpec((4, DW), lambda i: (0, 0)),
            pl.BlockSpec((1, DW), lambda i: (0, 0)),
        ],
        out_specs=pl.BlockSpec((be, DW), lambda i: (i, 0)),
        out_shape=jax.ShapeDtypeStruct((N_EDGES, DW), _f32),
    )(hedge, w2e, b2e, ccat, b1cat)


def _edge_out_body(h_ref, w2_ref, b2_ref, out_ref):
    out_ref[...] = _dot(h_ref[...], w2_ref[...]) + b2_ref[...]


def _edge_out(hedge, w2e, b2e):
    be = 2000
    grid = (N_EDGES // be,)
    return pl.pallas_call(
        _edge_out_body,
        grid=grid,
        in_specs=[
            pl.BlockSpec((be, HID), lambda i: (i, 0)),
            pl.BlockSpec((HID, 4), lambda i: (0, 0)),
            pl.BlockSpec((1, 4), lambda i: (0, 0)),
        ],
        out_specs=pl.BlockSpec((be, 4), lambda i: (i, 0)),
        out_shape=jax.ShapeDtypeStruct((N_EDGES, 4), _f32),
    )(hedge, w2e, b2e)


def _post_body_nofinal(s_ref, d_ref, x_ref, w2m_ref, b2m_ref, u1a_ref,
                       u1b_ref, bu1_ref, u2_ref, bu2_ref, out_ref):
    st = s_ref[0] + s_ref[1]
    deg = d_ref[0, :, 0:1] + d_ref[1, :, 0:1]
    agg = _dot(st, w2m_ref[...]) + deg * b2m_ref[...]
    u = jnp.maximum(
        _dot(x_ref[...], u1a_ref[...]) + _dot(agg, u1b_ref[...]) + bu1_ref[...],
        0.0)
    out_ref[...] = _dot(u, u2_ref[...]) + bu2_ref[...]


def _post_body_final(s_ref, d_ref, x_ref, w2m_ref, b2m_ref, u1a_ref,
                     u1b_ref, bu1_ref, u2_ref, bu2_ref, f1_ref, fb1_ref,
                     f2_ref, fb2_ref, out_ref):
    st = s_ref[0] + s_ref[1]
    deg = d_ref[0, :, 0:1] + d_ref[1, :, 0:1]
    agg = _dot(st, w2m_ref[...]) + deg * b2m_ref[...]
    u = jnp.maximum(
        _dot(x_ref[...], u1a_ref[...]) + _dot(agg, u1b_ref[...]) + bu1_ref[...],
        0.0)
    x2 = _dot(u, u2_ref[...]) + bu2_ref[...]
    y = jnp.maximum(_dot(x2, f1_ref[...]) + fb1_ref[...], 0.0)
    out_ref[...] = _dot(y, f2_ref[...]) + fb2_ref[...]


def _post(s, d, x, w2m, b2m, u1a, u1b, bu1, u2, bu2, fin=None):
    bn = 1000
    grid = (N_NODES // bn,)
    mat = lambda r, c: pl.BlockSpec((r, c), lambda i: (0, 0))
    in_specs = [
        pl.BlockSpec((NC, bn, SW), lambda i: (0, i, 0)),
        pl.BlockSpec((NC, bn, SW), lambda i: (0, i, 0)),
        pl.BlockSpec((bn, HID), lambda i: (i, 0)),
        mat(HID, HID), mat(1, HID),
        mat(HID, HID), mat(HID, HID), mat(1, HID),
        mat(HID, HID), mat(1, HID),
    ]
    args = [s, d, x, w2m, b2m, u1a, u1b, bu1, u2, bu2]
    body = _post_body_nofinal
    if fin is not None:
        f1, fb1, f2, fb2 = fin
        in_specs += [mat(HID, HID), mat(1, HID), mat(HID, HID), mat(1, HID)]
        args += [f1, fb1, f2, fb2]
        body = _post_body_final
    return pl.pallas_call(
        body,
        grid=grid,
        in_specs=in_specs,
        out_specs=pl.BlockSpec((bn, HID), lambda i: (i, 0)),
        out_shape=jax.ShapeDtypeStruct((N_NODES, HID), _f32),
    )(*args)


def _sc_edge_kernel(ps_hbm, pd_hbm, e_hbm, src_hbm, dst_hbm,
                    s_out, he_out,
                    srcv, dstv, gs, gd, ec, hm, he, S,
                    sem1, sem2, sem3):
    cid = lax.axis_index("c")
    sid = lax.axis_index("s")

    def zrow(i, carry):
        for j in range(SW // 16):
            hm[i, pl.ds(j * 16, 16)] = jnp.zeros((16,), _f32)
        return carry

    lax.fori_loop(0, ZR, zrow, 0)

    def zcopy(c, carry):
        pltpu.sync_copy(hm, S.at[pl.ds((c * NS + sid) * ZR, ZR), :])
        return carry

    lax.fori_loop(0, NZC // NS, zcopy, 0)

    @pl.when(sid < NZC % NS)
    def _():
        pltpu.sync_copy(hm, S.at[pl.ds(((NZC // NS) * NS + sid) * ZR, ZR), :])

    plsc.subcore_barrier()

    wid = sid * NC + cid
    ebase = wid * EPW

    def chunk(c, carry):
        start = ebase + c * B
        pltpu.sync_copy(src_hbm.at[pl.ds(start, B)], srcv)
        pltpu.sync_copy(dst_hbm.at[pl.ds(start, B)], dstv)
        cp1 = pltpu.async_copy(ps_hbm.at[srcv], gs, sem1)
        cp2 = pltpu.async_copy(pd_hbm.at[dstv], gd, sem2)
        cp3 = pltpu.async_copy(e_hbm.at[pl.ds(start, B), :], ec, sem3)
        cp1.wait()
        cp2.wait()
        cp3.wait()

        def edge(i, icarry):
            for j in range(HID // 16):
                o = j * 16
                hm[i, pl.ds(o, 16)] = jnp.maximum(
                    gs[i, pl.ds(o, 16)] + gd[i, pl.ds(o, 16)]
                    + ec[i, pl.ds(o, 16)], 0.0)
            for j in range(HID // 16):
                o = HID + j * 16
                he[i, pl.ds(j * 16, 16)] = jnp.maximum(
                    gs[i, pl.ds(o, 16)] + gd[i, pl.ds(o, 16)]
                    + ec[i, pl.ds(o, 16)], 0.0)
            return icarry

        lax.fori_loop(0, B, edge, 0)
        pltpu.sync_copy(he, he_out.at[pl.ds(start, B), :])
        pltpu.sync_copy(hm, S.at[dstv], add=True)
        return carry

    lax.fori_loop(0, NCHUNK, chunk, 0)
    plsc.subcore_barrier()

    def wback(c, carry):
        base = (c * NS + sid) * ZR
        pltpu.sync_copy(S.at[pl.ds(base, ZR), :],
                        s_out.at[cid, pl.ds(base, ZR), :])
        return carry

    lax.fori_loop(0, NZC // NS, wback, 0)

    @pl.when(sid < NZC % NS)
    def _():
        base = ((NZC // NS) * NS + sid) * ZR
        pltpu.sync_copy(S.at[pl.ds(base, ZR), :],
                        s_out.at[cid, pl.ds(base, ZR), :])


def _sc_edge(psrc, pdst, e, src, dst):
    mesh = plsc.VectorSubcoreMesh(core_axis_name="c", subcore_axis_name="s")
    fn = functools.partial(
        pl.kernel,
        mesh=mesh,
        out_type=(
            jax.ShapeDtypeStruct((NC, N_NODES, SW), _f32),
            jax.ShapeDtypeStruct((N_EDGES, HID), _f32),
        ),
        scratch_types=[
            pltpu.VMEM((B,), jnp.int32),
            pltpu.VMEM((B,), jnp.int32),
            pltpu.VMEM((B, DW), _f32),
            pltpu.VMEM((B, DW), _f32),
            pltpu.VMEM((B, DW), _f32),
            pltpu.VMEM((B, SW), _f32),
            pltpu.VMEM((B, HID), _f32),
            pltpu.VMEM_SHARED((N_NODES, SW), _f32),
            pltpu.SemaphoreType.DMA,
            pltpu.SemaphoreType.DMA,
            pltpu.SemaphoreType.DMA,
        ],
    )(_sc_edge_kernel)
    return fn(psrc, pdst, e, src, dst)


def _sc_deg_kernel(dst_hbm, d_out, dstv, onesb, zbuf, S, sem1):
    cid = lax.axis_index("c")
    sid = lax.axis_index("s")

    def zrow(i, carry):
        for j in range(SW // 16):
            zbuf[i, pl.ds(j * 16, 16)] = jnp.zeros((16,), _f32)
        return carry

    lax.fori_loop(0, ZR, zrow, 0)

    lane = lax.iota(jnp.int32, 16)
    onesv = jnp.where(lane == 0, jnp.full((16,), 1.0, _f32),
                      jnp.zeros((16,), _f32))

    def orow(i, carry):
        onesb[i, pl.ds(0, 16)] = onesv
        for j in range(1, SW // 16):
            onesb[i, pl.ds(j * 16, 16)] = jnp.zeros((16,), _f32)
        return carry

    lax.fori_loop(0, B, orow, 0)

    def zcopy(c, carry):
        pltpu.sync_copy(zbuf, S.at[pl.ds((c * NS + sid) * ZR, ZR), :])
        return carry

    lax.fori_loop(0, NZC // NS, zcopy, 0)

    @pl.when(sid < NZC % NS)
    def _():
        pltpu.sync_copy(zbuf, S.at[pl.ds(((NZC // NS) * NS + sid) * ZR, ZR), :])

    plsc.subcore_barrier()

    wid = sid * NC + cid
    ebase = wid * EPW

    def chunk(c, carry):
        start = ebase + c * B
        pltpu.sync_copy(dst_hbm.at[pl.ds(start, B)], dstv)
        pltpu.sync_copy(onesb, S.at[dstv], add=True)
        return carry

    lax.fori_loop(0, NCHUNK, chunk, 0)
    plsc.subcore_barrier()

    def wback(c, carry):
        base = (c * NS + sid) * ZR
        pltpu.sync_copy(S.at[pl.ds(base, ZR), :],
                        d_out.at[cid, pl.ds(base, ZR), :])
        return carry

    lax.fori_loop(0, NZC // NS, wback, 0)

    @pl.when(sid < NZC % NS)
    def _():
        base = ((NZC // NS) * NS + sid) * ZR
        pltpu.sync_copy(S.at[pl.ds(base, ZR), :],
                        d_out.at[cid, pl.ds(base, ZR), :])


def _sc_deg(dst):
    mesh = plsc.VectorSubcoreMesh(core_axis_name="c", subcore_axis_name="s")
    fn = functools.partial(
        pl.kernel,
        mesh=mesh,
        out_type=jax.ShapeDtypeStruct((NC, N_NODES, SW), _f32),
        scratch_types=[
            pltpu.VMEM((B,), jnp.int32),
            pltpu.VMEM((B, SW), _f32),
            pltpu.VMEM((ZR, SW), _f32),
            pltpu.VMEM_SHARED((N_NODES, SW), _f32),
            pltpu.SemaphoreType.DMA,
        ],
    )(_sc_deg_kernel)
    return fn(dst)


def kernel(nodes, edge_index, edge_attr, params):
    src = edge_index[0]
    dst = edge_index[1]

    def layer_weights(lp):
        wm1, we1 = lp["msg"]["W1"], lp["edge"]["W1"]
        wsrc = jnp.concatenate([wm1[0:HID], we1[0:HID]], axis=1)
        wdst = jnp.concatenate([wm1[HID:2 * HID], we1[HID:2 * HID]], axis=1)
        ccat = jnp.concatenate([wm1[2 * HID:], we1[2 * HID:]], axis=1)
        b1cat = jnp.concatenate([lp["msg"]["b1"], lp["edge"]["b1"]])
        return wsrc, wdst, ccat, b1cat.reshape(1, DW)

    l1, l2 = params["layers"]
    ws1, wd1, c1, b1c1 = layer_weights(l1)
    ws2, wd2, c2, b1c2 = layer_weights(l2)
    row = lambda v: v.reshape(1, -1)

    deg = _sc_deg(dst)

    psrc1, pdst1 = _proj(nodes, ws1, wd1)
    e1 = _edge_pre(edge_attr, c1, b1c1)
    s1, hedge1 = _sc_edge(psrc1, pdst1, e1, src, dst)
    nodes2 = _post(s1, deg, nodes,
                   l1["msg"]["W2"], row(l1["msg"]["b2"]),
                   l1["upd"]["W1"][:HID], l1["upd"]["W1"][HID:],
                   row(l1["upd"]["b1"]), l1["upd"]["W2"], row(l1["upd"]["b2"]))

    psrc2, pdst2 = _proj(nodes2, ws2, wd2)
    e2 = _edge_pre2(hedge1, l1["edge"]["W2"], row(l1["edge"]["b2"]), c2, b1c2)
    s2, hedge2 = _sc_edge(psrc2, pdst2, e2, src, dst)
    fp = params["final"]
    out_nodes = _post(s2, deg, nodes2,
                      l2["msg"]["W2"], row(l2["msg"]["b2"]),
                      l2["upd"]["W1"][:HID], l2["upd"]["W1"][HID:],
                      row(l2["upd"]["b1"]), l2["upd"]["W2"],
                      row(l2["upd"]["b2"]),
                      fin=(fp["W1"], row(fp["b1"]), fp["W2"], row(fp["b2"])))
    out_edge_attr = _edge_out(hedge2, l2["edge"]["W2"], row(l2["edge"]["b2"]))
    return out_nodes, out_edge_attr


# speedup vs baseline: 1.6648x; 1.4894x over previous
import functools

import jax
import jax.numpy as jnp
from jax import lax
from jax.experimental import pallas as pl
from jax.experimental.pallas import tpu as pltpu
from jax.experimental.pallas import tpu_sc as plsc

N_NODES = 10000
N_EDGES = 320000
HID = 128
DW = 256
SW = 128
NC = 2
NS = 16
NW = NC * NS
EPW = N_EDGES // NW
B = 40
NCHUNK = EPW // B
ZR = 40
NZC = N_NODES // ZR

_f32 = jnp.float32


def _dot(a, b):
    return jnp.dot(a, b, preferred_element_type=_f32)


def _proj_body(x_ref, ws_ref, wd_ref, ps_ref, pd_ref):
    x = x_ref[...]
    ps_ref[...] = _dot(x, ws_ref[...])
    pd_ref[...] = _dot(x, wd_ref[...])


def _proj(x, wsrc, wdst):
    bn = 1000
    grid = (N_NODES // bn,)
    return pl.pallas_call(
        _proj_body,
        grid=grid,
        in_specs=[
            pl.BlockSpec((bn, HID), lambda i: (i, 0)),
            pl.BlockSpec((HID, DW), lambda i: (0, 0)),
            pl.BlockSpec((HID, DW), lambda i: (0, 0)),
        ],
        out_specs=[
            pl.BlockSpec((bn, DW), lambda i: (i, 0)),
            pl.BlockSpec((bn, DW), lambda i: (i, 0)),
        ],
        out_shape=[
            jax.ShapeDtypeStruct((N_NODES, DW), _f32),
            jax.ShapeDtypeStruct((N_NODES, DW), _f32),
        ],
    )(x, wsrc, wdst)


def _edge_pre_body(ea_ref, c_ref, b_ref, out_ref):
    out_ref[...] = _dot(ea_ref[...], c_ref[...]) + b_ref[...]


def _edge_pre(ea, ccat, b1cat):
    be = 2000
    grid = (N_EDGES // be,)
    return pl.pallas_call(
        _edge_pre_body,
        grid=grid,
        in_specs=[
            pl.BlockSpec((be, 4), lambda i: (i, 0)),
            pl.BlockSpec((4, DW), lambda i: (0, 0)),
            pl.BlockSpec((1, DW), lambda i: (0, 0)),
        ],
        out_specs=pl.BlockSpec((be, DW), lambda i: (i, 0)),
        out_shape=jax.ShapeDtypeStruct((N_EDGES, DW), _f32),
    )(ea, ccat, b1cat)


def _edge_pre2_body(h_ref, w2_ref, b2_ref, c_ref, b_ref, out_ref):
    ea = _dot(h_ref[...], w2_ref[...]) + b2_ref[...]
    out_ref[...] = _dot(ea, c_ref[...]) + b_ref[...]


def _edge_pre2(hedge, w2e, b2e, ccat, b1cat):
    be = 2000
    grid = (N_EDGES // be,)
    return pl.pallas_call(
        _edge_pre2_body,
        grid=grid,
        in_specs=[
            pl.BlockSpec((be, HID), lambda i: (i, 0)),
            pl.BlockSpec((HID, 4), lambda i: (0, 0)),
            pl.BlockSpec((1, 4), lambda i: (0, 0)),
            pl.BlockSpec((4, DW), lambda i: (0, 0)),
            pl.BlockSpec((1, DW), lambda i: (0, 0)),
        ],
        out_specs=pl.BlockSpec((be, DW), lambda i: (i, 0)),
        out_shape=jax.ShapeDtypeStruct((N_EDGES, DW), _f32),
    )(hedge, w2e, b2e, ccat, b1cat)


def _edge_out_body(h_ref, w2_ref, b2_ref, out_ref):
    out_ref[...] = _dot(h_ref[...], w2_ref[...]) + b2_ref[...]


def _edge_out(hedge, w2e, b2e):
    be = 2000
    grid = (N_EDGES // be,)
    return pl.pallas_call(
        _edge_out_body,
        grid=grid,
        in_specs=[
            pl.BlockSpec((be, HID), lambda i: (i, 0)),
            pl.BlockSpec((HID, 4), lambda i: (0, 0)),
            pl.BlockSpec((1, 4), lambda i: (0, 0)),
        ],
        out_specs=pl.BlockSpec((be, 4), lambda i: (i, 0)),
        out_shape=jax.ShapeDtypeStruct((N_EDGES, 4), _f32),
    )(hedge, w2e, b2e)


def _post_body_nofinal(s_ref, d_ref, x_ref, w2m_ref, b2m_ref, u1a_ref,
                       u1b_ref, bu1_ref, u2_ref, bu2_ref, out_ref):
    st = s_ref[0] + s_ref[1]
    deg = d_ref[0, :, 0:1] + d_ref[1, :, 0:1]
    agg = _dot(st, w2m_ref[...]) + deg * b2m_ref[...]
    u = jnp.maximum(
        _dot(x_ref[...], u1a_ref[...]) + _dot(agg, u1b_ref[...]) + bu1_ref[...],
        0.0)
    out_ref[...] = _dot(u, u2_ref[...]) + bu2_ref[...]


def _post_body_final(s_ref, d_ref, x_ref, w2m_ref, b2m_ref, u1a_ref,
                     u1b_ref, bu1_ref, u2_ref, bu2_ref, f1_ref, fb1_ref,
                     f2_ref, fb2_ref, out_ref):
    st = s_ref[0] + s_ref[1]
    deg = d_ref[0, :, 0:1] + d_ref[1, :, 0:1]
    agg = _dot(st, w2m_ref[...]) + deg * b2m_ref[...]
    u = jnp.maximum(
        _dot(x_ref[...], u1a_ref[...]) + _dot(agg, u1b_ref[...]) + bu1_ref[...],
        0.0)
    x2 = _dot(u, u2_ref[...]) + bu2_ref[...]
    y = jnp.maximum(_dot(x2, f1_ref[...]) + fb1_ref[...], 0.0)
    out_ref[...] = _dot(y, f2_ref[...]) + fb2_ref[...]


def _post(s, d, x, w2m, b2m, u1a, u1b, bu1, u2, bu2, fin=None):
    bn = 1000
    grid = (N_NODES // bn,)
    mat = lambda r, c: pl.BlockSpec((r, c), lambda i: (0, 0))
    in_specs = [
        pl.BlockSpec((NC, bn, SW), lambda i: (0, i, 0)),
        pl.BlockSpec((NC, bn, SW), lambda i: (0, i, 0)),
        pl.BlockSpec((bn, HID), lambda i: (i, 0)),
        mat(HID, HID), mat(1, HID),
        mat(HID, HID), mat(HID, HID), mat(1, HID),
        mat(HID, HID), mat(1, HID),
    ]
    args = [s, d, x, w2m, b2m, u1a, u1b, bu1, u2, bu2]
    body = _post_body_nofinal
    if fin is not None:
        f1, fb1, f2, fb2 = fin
        in_specs += [mat(HID, HID), mat(1, HID), mat(HID, HID), mat(1, HID)]
        args += [f1, fb1, f2, fb2]
        body = _post_body_final
    return pl.pallas_call(
        body,
        grid=grid,
        in_specs=in_specs,
        out_specs=pl.BlockSpec((bn, HID), lambda i: (i, 0)),
        out_shape=jax.ShapeDtypeStruct((N_NODES, HID), _f32),
    )(*args)


def _sc_edge_kernel(ps_hbm, pd_hbm, e_hbm, src_hbm, dst_hbm,
                    s_out, he_out,
                    srcv, dstv, gs, gd, ec, hm, he, S,
                    sem1, sem2, sem3):
    cid = lax.axis_index("c")
    sid = lax.axis_index("s")

    def zrow(i, carry):
        for j in range(SW // 16):
            hm[i, pl.ds(j * 16, 16)] = jnp.zeros((16,), _f32)
        return carry

    lax.fori_loop(0, ZR, zrow, 0)

    def zcopy(c, carry):
        pltpu.sync_copy(hm, S.at[pl.ds((c * NS + sid) * ZR, ZR), :])
        return carry

    lax.fori_loop(0, NZC // NS, zcopy, 0)

    @pl.when(sid < NZC % NS)
    def _():
        pltpu.sync_copy(hm, S.at[pl.ds(((NZC // NS) * NS + sid) * ZR, ZR), :])

    plsc.subcore_barrier()

    wid = sid * NC + cid
    ebase = wid * EPW

    def chunk(c, carry):
        start = ebase + c * B
        pltpu.sync_copy(src_hbm.at[pl.ds(start, B)], srcv)
        pltpu.sync_copy(dst_hbm.at[pl.ds(start, B)], dstv)
        cp1 = pltpu.async_copy(ps_hbm.at[srcv], gs, sem1)
        cp2 = pltpu.async_copy(pd_hbm.at[dstv], gd, sem2)
        cp3 = pltpu.async_copy(e_hbm.at[pl.ds(start, B), :], ec, sem3)
        cp1.wait()
        cp2.wait()
        cp3.wait()

        @plsc.parallel_loop(0, B, 1, unroll=4)
        def _(i):
            for j in range(HID // 16):
                o = j * 16
                hm[i, pl.ds(o, 16)] = jnp.maximum(
                    gs[i, pl.ds(o, 16)] + gd[i, pl.ds(o, 16)]
                    + ec[i, pl.ds(o, 16)], 0.0)
            for j in range(HID // 16):
                o = HID + j * 16
                he[i, pl.ds(j * 16, 16)] = jnp.maximum(
                    gs[i, pl.ds(o, 16)] + gd[i, pl.ds(o, 16)]
                    + ec[i, pl.ds(o, 16)], 0.0)
        pltpu.sync_copy(he, he_out.at[pl.ds(start, B), :])
        pltpu.sync_copy(hm, S.at[dstv], add=True)
        return carry

    lax.fori_loop(0, NCHUNK, chunk, 0)
    plsc.subcore_barrier()

    def wback(c, carry):
        base = (c * NS + sid) * ZR
        pltpu.sync_copy(S.at[pl.ds(base, ZR), :],
                        s_out.at[cid, pl.ds(base, ZR), :])
        return carry

    lax.fori_loop(0, NZC // NS, wback, 0)

    @pl.when(sid < NZC % NS)
    def _():
        base = ((NZC // NS) * NS + sid) * ZR
        pltpu.sync_copy(S.at[pl.ds(base, ZR), :],
                        s_out.at[cid, pl.ds(base, ZR), :])


def _sc_edge(psrc, pdst, e, src, dst):
    mesh = plsc.VectorSubcoreMesh(core_axis_name="c", subcore_axis_name="s")
    fn = functools.partial(
        pl.kernel,
        mesh=mesh,
        out_type=(
            jax.ShapeDtypeStruct((NC, N_NODES, SW), _f32),
            jax.ShapeDtypeStruct((N_EDGES, HID), _f32),
        ),
        scratch_types=[
            pltpu.VMEM((B,), jnp.int32),
            pltpu.VMEM((B,), jnp.int32),
            pltpu.VMEM((B, DW), _f32),
            pltpu.VMEM((B, DW), _f32),
            pltpu.VMEM((B, DW), _f32),
            pltpu.VMEM((B, SW), _f32),
            pltpu.VMEM((B, HID), _f32),
            pltpu.VMEM_SHARED((N_NODES, SW), _f32),
            pltpu.SemaphoreType.DMA,
            pltpu.SemaphoreType.DMA,
            pltpu.SemaphoreType.DMA,
        ],
    )(_sc_edge_kernel)
    return fn(psrc, pdst, e, src, dst)


def _sc_deg_kernel(dst_hbm, d_out, dstv, onesb, zbuf, S, sem1):
    cid = lax.axis_index("c")
    sid = lax.axis_index("s")

    def zrow(i, carry):
        for j in range(SW // 16):
            zbuf[i, pl.ds(j * 16, 16)] = jnp.zeros((16,), _f32)
        return carry

    lax.fori_loop(0, ZR, zrow, 0)

    lane = lax.iota(jnp.int32, 16)
    onesv = jnp.where(lane == 0, jnp.full((16,), 1.0, _f32),
                      jnp.zeros((16,), _f32))

    def orow(i, carry):
        onesb[i, pl.ds(0, 16)] = onesv
        for j in range(1, SW // 16):
            onesb[i, pl.ds(j * 16, 16)] = jnp.zeros((16,), _f32)
        return carry

    lax.fori_loop(0, B, orow, 0)

    def zcopy(c, carry):
        pltpu.sync_copy(zbuf, S.at[pl.ds((c * NS + sid) * ZR, ZR), :])
        return carry

    lax.fori_loop(0, NZC // NS, zcopy, 0)

    @pl.when(sid < NZC % NS)
    def _():
        pltpu.sync_copy(zbuf, S.at[pl.ds(((NZC // NS) * NS + sid) * ZR, ZR), :])

    plsc.subcore_barrier()

    wid = sid * NC + cid
    ebase = wid * EPW

    def chunk(c, carry):
        start = ebase + c * B
        pltpu.sync_copy(dst_hbm.at[pl.ds(start, B)], dstv)
        pltpu.sync_copy(onesb, S.at[dstv], add=True)
        return carry

    lax.fori_loop(0, NCHUNK, chunk, 0)
    plsc.subcore_barrier()

    def wback(c, carry):
        base = (c * NS + sid) * ZR
        pltpu.sync_copy(S.at[pl.ds(base, ZR), :],
                        d_out.at[cid, pl.ds(base, ZR), :])
        return carry

    lax.fori_loop(0, NZC // NS, wback, 0)

    @pl.when(sid < NZC % NS)
    def _():
        base = ((NZC // NS) * NS + sid) * ZR
        pltpu.sync_copy(S.at[pl.ds(base, ZR), :],
                        d_out.at[cid, pl.ds(base, ZR), :])


def _sc_deg(dst):
    mesh = plsc.VectorSubcoreMesh(core_axis_name="c", subcore_axis_name="s")
    fn = functools.partial(
        pl.kernel,
        mesh=mesh,
        out_type=jax.ShapeDtypeStruct((NC, N_NODES, SW), _f32),
        scratch_types=[
            pltpu.VMEM((B,), jnp.int32),
            pltpu.VMEM((B, SW), _f32),
            pltpu.VMEM((ZR, SW), _f32),
            pltpu.VMEM_SHARED((N_NODES, SW), _f32),
            pltpu.SemaphoreType.DMA,
        ],
    )(_sc_deg_kernel)
    return fn(dst)


def kernel(nodes, edge_index, edge_attr, params):
    src = edge_index[0]
    dst = edge_index[1]

    def layer_weights(lp):
        wm1, we1 = lp["msg"]["W1"], lp["edge"]["W1"]
        wsrc = jnp.concatenate([wm1[0:HID], we1[0:HID]], axis=1)
        wdst = jnp.concatenate([wm1[HID:2 * HID], we1[HID:2 * HID]], axis=1)
        ccat = jnp.concatenate([wm1[2 * HID:], we1[2 * HID:]], axis=1)
        b1cat = jnp.concatenate([lp["msg"]["b1"], lp["edge"]["b1"]])
        return wsrc, wdst, ccat, b1cat.reshape(1, DW)

    l1, l2 = params["layers"]
    ws1, wd1, c1, b1c1 = layer_weights(l1)
    ws2, wd2, c2, b1c2 = layer_weights(l2)
    row = lambda v: v.reshape(1, -1)

    deg = _sc_deg(dst)

    psrc1, pdst1 = _proj(nodes, ws1, wd1)
    e1 = _edge_pre(edge_attr, c1, b1c1)
    s1, hedge1 = _sc_edge(psrc1, pdst1, e1, src, dst)
    nodes2 = _post(s1, deg, nodes,
                   l1["msg"]["W2"], row(l1["msg"]["b2"]),
                   l1["upd"]["W1"][:HID], l1["upd"]["W1"][HID:],
                   row(l1["upd"]["b1"]), l1["upd"]["W2"], row(l1["upd"]["b2"]))

    psrc2, pdst2 = _proj(nodes2, ws2, wd2)
    e2 = _edge_pre2(hedge1, l1["edge"]["W2"], row(l1["edge"]["b2"]), c2, b1c2)
    s2, hedge2 = _sc_edge(psrc2, pdst2, e2, src, dst)
    fp = params["final"]
    out_nodes = _post(s2, deg, nodes2,
                      l2["msg"]["W2"], row(l2["msg"]["b2"]),
                      l2["upd"]["W1"][:HID], l2["upd"]["W1"][HID:],
                      row(l2["upd"]["b1"]), l2["upd"]["W2"],
                      row(l2["upd"]["b2"]),
                      fin=(fp["W1"], row(fp["b1"]), fp["W2"], row(fp["b2"])))
    out_edge_attr = _edge_out(hedge2, l2["edge"]["W2"], row(l2["edge"]["b2"]))
    return out_nodes, out_edge_attr
